# baseline (speedup 1.0000x reference)
import jax
import jax.numpy as jnp
from jax import lax
from jax.experimental import pallas as pl
from jax.experimental.pallas import tpu as pltpu
from jax.experimental.pallas import tpu_sc as plsc

_T = 100000
_V = 50000
_NW = 32
_C = 768
_GPC = _C // 16
_CHUNKS = 4
_MAIN_T = _NW * _CHUNKS * _C
_TAIL_GROUPS = (_T - _MAIN_T) // 16
_MAIN_G = _MAIN_T // 16


def _flatten_idx(idx2d_v, idxf_v, n_tri):
    iota = lax.iota(jnp.int32, 16)
    toff = [(16 * s + iota) // 3 for s in range(3)]
    coff = [(16 * s + iota) % 3 for s in range(3)]

    def body(u, carry):
        for s in range(3):
            v = plsc.load_gather(idx2d_v, [16 * u + toff[s], coff[s]])
            idxf_v[pl.ds(48 * u + 16 * s, 16)] = v
        return carry

    lax.fori_loop(0, n_tri // 16, body, 0)


def _compute_group(verts_v, bary_v, out_v, t0):
    iota = lax.iota(jnp.int32, 16)
    tv = t0 + iota
    r3 = 3 * tv
    ov = 6 * tv
    v = [[plsc.load_gather(verts_v, [r3 + k, jnp.full((16,), j, jnp.int32)])
          for j in range(3)] for k in range(3)]
    for p in range(6):
        pv = jnp.full((16,), p, jnp.int32)
        b = [plsc.load_gather(bary_v, [tv, pv, jnp.full((16,), i, jnp.int32)])
             for i in range(3)]
        r = 1.0 / (b[0] + b[1] + b[2])
        for j in range(3):
            acc = b[0] * v[0][j] + b[1] * v[1][j] + b[2] * v[2][j]
            plsc.store_scatter(out_v, [ov + p, jnp.full((16,), j, jnp.int32)],
                               acc * r)


def _mesh_body(tri_hbm, bary_hbm, table_hbm, out_hbm,
               idx2d_v, idxf_v, verts_v, bary_v, out_v, sem):
    wid = lax.axis_index("s") * 2 + lax.axis_index("c")

    def chunk_body(c, carry):
        t_base = (wid * _CHUNKS + c) * _C
        pltpu.sync_copy(tri_hbm.at[pl.ds(t_base, _C)], idx2d_v)
        _flatten_idx(idx2d_v, idxf_v, _C)
        copies = [
            pltpu.async_copy(table_hbm.at[idxf_v.at[pl.ds(128 * j, 128)]],
                             verts_v.at[pl.ds(128 * j, 128)], sem)
            for j in range(3 * _C // 128)
        ]
        for cp in copies:
            cp.wait()
        pltpu.sync_copy(bary_hbm.at[pl.ds(t_base, _C)], bary_v)

        def group_body(g, carry2):
            _compute_group(verts_v, bary_v, out_v, 16 * g)
            return carry2

        lax.fori_loop(0, _GPC, group_body, 0)
        pltpu.sync_copy(out_v, out_hbm.at[pl.ds(6 * t_base, 6 * _C)])
        return carry

    lax.fori_loop(0, _CHUNKS, chunk_body, 0)

    n_tail = jnp.where(wid < _TAIL_GROUPS - 3 * _NW, 4, 3)

    def tail_body(k, carry):
        t_base = 16 * (_MAIN_G + wid + _NW * k)
        pltpu.sync_copy(tri_hbm.at[pl.ds(t_base, 16)],
                        idx2d_v.at[pl.ds(0, 16)])
        _flatten_idx(idx2d_v, idxf_v, 16)
        pltpu.async_copy(table_hbm.at[idxf_v.at[pl.ds(0, 48)]],
                         verts_v.at[pl.ds(0, 48)], sem).wait()
        pltpu.sync_copy(bary_hbm.at[pl.ds(t_base, 16)],
                        bary_v.at[pl.ds(0, 16)])
        _compute_group(verts_v, bary_v, out_v, 0)
        pltpu.sync_copy(out_v.at[pl.ds(0, 96)],
                        out_hbm.at[pl.ds(6 * t_base, 96)])
        return carry

    lax.fori_loop(0, n_tail, tail_body, 0)


_mesh_kernel = pl.kernel(
    _mesh_body,
    out_type=jax.ShapeDtypeStruct((_T * 6, 3), jnp.float32),
    mesh=plsc.VectorSubcoreMesh(core_axis_name="c", subcore_axis_name="s"),
    scratch_types=[
        pltpu.VMEM((_C, 3), jnp.int32),
        pltpu.VMEM((3 * _C,), jnp.int32),
        pltpu.VMEM((3 * _C, 8), jnp.float32),
        pltpu.VMEM((_C, 6, 3), jnp.float32),
        pltpu.VMEM((6 * _C, 3), jnp.float32),
        pltpu.SemaphoreType.DMA,
    ],
    compiler_params=pltpu.CompilerParams(
        needs_layout_passes=False, use_tc_tiling_on_sc=False),
)


def kernel(vertex_coords, bary_coords, triangles):
    table8 = jnp.pad(vertex_coords, ((0, 0), (0, 5)))
    return _mesh_kernel(triangles, bary_coords, table8)


# speedup vs baseline: 12.5652x; 12.5652x over previous
import jax
import jax.numpy as jnp
from jax import lax
from jax.experimental import pallas as pl
from jax.experimental.pallas import tpu as pltpu
from jax.experimental.pallas import tpu_sc as plsc

_T = 100000
_V = 50000
_NW = 32
_C = 768
_GPC = _C // 16
_CHUNKS = 4
_MAIN_T = _NW * _CHUNKS * _C
_TAIL_GROUPS = (_T - _MAIN_T) // 16
_MAIN_G = _MAIN_T // 16


def _compute_group(verts_v, bary_v, out_v, t0):
    iota = lax.iota(jnp.int32, 16)
    tv = t0 + iota
    ov = 6 * tv
    cols = [jnp.full((16,), j, jnp.int32) for j in range(3)]
    v = [[plsc.load_gather(verts_v[k], [tv, cols[j]]) for j in range(3)]
         for k in range(3)]
    for p in range(6):
        b = [bary_v[3 * p + i, pl.ds(t0, 16)] for i in range(3)]
        r = 1.0 / (b[0] + b[1] + b[2])
        for j in range(3):
            acc = b[0] * v[0][j] + b[1] * v[1][j] + b[2] * v[2][j]
            plsc.store_scatter(out_v, [cols[j], ov + p], acc * r)


def _mesh_body(tri_hbm, bary_hbm, table_hbm, out_hbm,
               idx_v, verts_v0, verts_v1, verts_v2, bary_v, out_v, sem):
    wid = lax.axis_index("s") * 2 + lax.axis_index("c")
    verts_v = [verts_v0, verts_v1, verts_v2]

    def chunk_body(c, carry):
        t_base = (wid * _CHUNKS + c) * _C
        for k in range(3):
            pltpu.sync_copy(tri_hbm.at[k, pl.ds(t_base, _C)], idx_v.at[k])
        copies = [
            pltpu.async_copy(
                table_hbm.at[idx_v.at[k, pl.ds(128 * j, 128)]],
                verts_v[k].at[pl.ds(128 * j, 128)], sem)
            for k in range(3) for j in range(_C // 128)
        ]
        for cp in copies:
            cp.wait()
        for p in range(6):
            for i in range(3):
                pltpu.sync_copy(bary_hbm.at[p, i, pl.ds(t_base, _C)],
                                bary_v.at[3 * p + i])

        def group_body(g, carry2):
            _compute_group(verts_v, bary_v, out_v, 16 * g)
            return carry2

        lax.fori_loop(0, _GPC, group_body, 0)
        for j in range(3):
            pltpu.sync_copy(out_v.at[j],
                            out_hbm.at[j, pl.ds(6 * t_base, 6 * _C)])
        return carry

    lax.fori_loop(0, _CHUNKS, chunk_body, 0)

    n_tail = jnp.where(wid < _TAIL_GROUPS - 3 * _NW, 4, 3)

    def tail_body(k_loop, carry):
        t_base = 16 * (_MAIN_G + wid + _NW * k_loop)
        for k in range(3):
            pltpu.sync_copy(tri_hbm.at[k, pl.ds(t_base, 16)],
                            idx_v.at[k, pl.ds(0, 16)])
            pltpu.async_copy(
                table_hbm.at[idx_v.at[k, pl.ds(0, 16)]],
                verts_v[k].at[pl.ds(0, 16)], sem).wait()
        for p in range(6):
            for i in range(3):
                pltpu.sync_copy(bary_hbm.at[p, i, pl.ds(t_base, 16)],
                                bary_v.at[3 * p + i, pl.ds(0, 16)])
        _compute_group(verts_v, bary_v, out_v, 0)
        for j in range(3):
            pltpu.sync_copy(out_v.at[j, pl.ds(0, 96)],
                            out_hbm.at[j, pl.ds(6 * t_base, 96)])
        return carry

    lax.fori_loop(0, n_tail, tail_body, 0)


_mesh_kernel = pl.kernel(
    _mesh_body,
    out_type=jax.ShapeDtypeStruct((3, _T * 6), jnp.float32),
    mesh=plsc.VectorSubcoreMesh(core_axis_name="c", subcore_axis_name="s"),
    scratch_types=[
        pltpu.VMEM((3, _C), jnp.int32),
        pltpu.VMEM((_C, 8), jnp.float32),
        pltpu.VMEM((_C, 8), jnp.float32),
        pltpu.VMEM((_C, 8), jnp.float32),
        pltpu.VMEM((18, _C), jnp.float32),
        pltpu.VMEM((3, 6 * _C), jnp.float32),
        pltpu.SemaphoreType.DMA,
    ],
    compiler_params=pltpu.CompilerParams(
        needs_layout_passes=False, use_tc_tiling_on_sc=False),
)


def kernel(vertex_coords, bary_coords, triangles):
    table8 = jnp.pad(vertex_coords, ((0, 0), (0, 5)))
    tri_t = jnp.transpose(triangles, (1, 0))
    bary_t = jnp.transpose(bary_coords, (1, 2, 0))
    out_t = _mesh_kernel(tri_t, bary_t, table8)
    return jnp.transpose(out_t, (1, 0))


# speedup vs baseline: 19.2185x; 1.5295x over previous
import jax
import jax.numpy as jnp
from jax import lax
from jax.experimental import pallas as pl
from jax.experimental.pallas import tpu as pltpu
from jax.experimental.pallas import tpu_sc as plsc

_T = 100000
_V = 50000
_NW = 32
_C = 768
_GPC = _C // 16
_CHUNKS = 4
_MAIN_T = _NW * _CHUNKS * _C
_TAIL_GROUPS = (_T - _MAIN_T) // 16
_MAIN_G = _MAIN_T // 16


def _compute_group(verts_v, bary_v, out_v, t0):
    iota = lax.iota(jnp.int32, 16)
    tv = t0 + iota
    ov = 6 * tv
    cols = [jnp.full((16,), j, jnp.int32) for j in range(3)]
    v = [[plsc.load_gather(verts_v[k], [tv, cols[j]]) for j in range(3)]
         for k in range(3)]
    for p in range(6):
        b = [bary_v[p, i, pl.ds(t0, 16)] for i in range(3)]
        r = 1.0 / (b[0] + b[1] + b[2])
        for j in range(3):
            acc = b[0] * v[0][j] + b[1] * v[1][j] + b[2] * v[2][j]
            plsc.store_scatter(out_v, [cols[j], ov + p], acc * r)


def _mesh_body(tri_hbm, bary_hbm, table_hbm, out_hbm,
               idx_v, verts_v0, verts_v1, verts_v2, bary_v, out_v,
               sem, bsem):
    wid = lax.axis_index("s") * 2 + lax.axis_index("c")
    verts_v = [verts_v0, verts_v1, verts_v2]

    def chunk_body(c, carry):
        t_base = (wid * _CHUNKS + c) * _C
        bary_cp = pltpu.async_copy(bary_hbm.at[:, :, pl.ds(t_base, _C)],
                                   bary_v, bsem)
        pltpu.sync_copy(tri_hbm.at[:, pl.ds(t_base, _C)], idx_v)
        copies = [
            pltpu.async_copy(
                table_hbm.at[idx_v.at[k, pl.ds(128 * j, 128)]],
                verts_v[k].at[pl.ds(128 * j, 128)], sem)
            for k in range(3) for j in range(_C // 128)
        ]
        for cp in copies:
            cp.wait()
        bary_cp.wait()

        def group_body(g, carry2):
            _compute_group(verts_v, bary_v, out_v, 16 * g)
            return carry2

        lax.fori_loop(0, _GPC, group_body, 0)
        pltpu.sync_copy(out_v, out_hbm.at[:, pl.ds(6 * t_base, 6 * _C)])
        return carry

    lax.fori_loop(0, _CHUNKS, chunk_body, 0)

    n_tail = jnp.where(wid < _TAIL_GROUPS - 3 * _NW, 4, 3)

    def tail_body(k_loop, carry):
        t_base = 16 * (_MAIN_G + wid + _NW * k_loop)
        pltpu.sync_copy(tri_hbm.at[:, pl.ds(t_base, 16)],
                        idx_v.at[:, pl.ds(0, 16)])
        for k in range(3):
            pltpu.async_copy(
                table_hbm.at[idx_v.at[k, pl.ds(0, 16)]],
                verts_v[k].at[pl.ds(0, 16)], sem).wait()
        pltpu.sync_copy(bary_hbm.at[:, :, pl.ds(t_base, 16)],
                        bary_v.at[:, :, pl.ds(0, 16)])
        _compute_group(verts_v, bary_v, out_v, 0)
        pltpu.sync_copy(out_v.at[:, pl.ds(0, 96)],
                        out_hbm.at[:, pl.ds(6 * t_base, 96)])
        return carry

    lax.fori_loop(0, n_tail, tail_body, 0)


_mesh_kernel = pl.kernel(
    _mesh_body,
    out_type=jax.ShapeDtypeStruct((3, _T * 6), jnp.float32),
    mesh=plsc.VectorSubcoreMesh(core_axis_name="c", subcore_axis_name="s"),
    scratch_types=[
        pltpu.VMEM((3, _C), jnp.int32),
        pltpu.VMEM((_C, 8), jnp.float32),
        pltpu.VMEM((_C, 8), jnp.float32),
        pltpu.VMEM((_C, 8), jnp.float32),
        pltpu.VMEM((6, 3, _C), jnp.float32),
        pltpu.VMEM((3, 6 * _C), jnp.float32),
        pltpu.SemaphoreType.DMA,
        pltpu.SemaphoreType.DMA,
    ],
    compiler_params=pltpu.CompilerParams(
        needs_layout_passes=False, use_tc_tiling_on_sc=False),
)


def kernel(vertex_coords, bary_coords, triangles):
    table8 = jnp.pad(vertex_coords, ((0, 0), (0, 5)))
    tri_t = jnp.transpose(triangles, (1, 0))
    bary_t = jnp.transpose(bary_coords, (1, 2, 0))
    out_t = _mesh_kernel(tri_t, bary_t, table8)
    return jnp.transpose(out_t, (1, 0))


# speedup vs baseline: 20.3939x; 1.0612x over previous
import jax
import jax.numpy as jnp
from jax import lax
from jax.experimental import pallas as pl
from jax.experimental.pallas import tpu as pltpu
from jax.experimental.pallas import tpu_sc as plsc

_T = 100000
_V = 50000
_NW = 32
_C = 384
_GPC = _C // 16
_CHUNKS = 8
_MAIN_T = _NW * _CHUNKS * _C
_TAIL_GROUPS = (_T - _MAIN_T) // 16
_MAIN_G = _MAIN_T // 16


def _compute_group(verts_v, bary_v, out_v, t0):
    iota = lax.iota(jnp.int32, 16)
    tv = t0 + iota
    ov = 6 * tv
    cols = [jnp.full((16,), j, jnp.int32) for j in range(3)]
    v = [[plsc.load_gather(verts_v[k], [tv, cols[j]]) for j in range(3)]
         for k in range(3)]
    for p in range(6):
        b = [bary_v[p, i, pl.ds(t0, 16)] for i in range(3)]
        r = 1.0 / (b[0] + b[1] + b[2])
        for j in range(3):
            acc = b[0] * v[0][j] + b[1] * v[1][j] + b[2] * v[2][j]
            plsc.store_scatter(out_v, [cols[j], ov + p], acc * r)


def _mesh_body(tri_hbm, bary_hbm, table_hbm, out_hbm,
               idx_v0, idx_v1, va0, vb0, vc0, va1, vb1, vc1,
               bary_v0, bary_v1, out_v0, out_v1, sem, bsem, osem):
    wid = lax.axis_index("s") * 2 + lax.axis_index("c")
    idx_v = [idx_v0, idx_v1]
    verts_v = [[va0, vb0, vc0], [va1, vb1, vc1]]
    bary_v = [bary_v0, bary_v1]
    out_v = [out_v0, out_v1]

    def start_in(s, c):
        t_base = (wid * _CHUNKS + c) * _C
        bary_cp = pltpu.async_copy(bary_hbm.at[:, :, pl.ds(t_base, _C)],
                                   bary_v[s], bsem)
        pltpu.sync_copy(tri_hbm.at[:, pl.ds(t_base, _C)], idx_v[s])
        gathers = [
            pltpu.async_copy(
                table_hbm.at[idx_v[s].at[k, pl.ds(128 * j, 128)]],
                verts_v[s][k].at[pl.ds(128 * j, 128)], sem)
            for k in range(3) for j in range(_C // 128)
        ]
        return gathers + [bary_cp]

    in_cps = {0: start_in(0, 0)}
    out_cps = {}
    for u in range(_CHUNKS):
        s = u & 1
        if u + 1 < _CHUNKS:
            in_cps[u + 1] = start_in((u + 1) & 1, u + 1)
        for cp in in_cps.pop(u):
            cp.wait()
        if u >= 2:
            out_cps.pop(u - 2).wait()

        def group_body(g, carry2, s=s):
            _compute_group(verts_v[s], bary_v[s], out_v[s], 16 * g)
            return carry2

        lax.fori_loop(0, _GPC, group_body, 0)
        t_base = (wid * _CHUNKS + u) * _C
        out_cps[u] = pltpu.async_copy(
            out_v[s], out_hbm.at[:, pl.ds(6 * t_base, 6 * _C)], osem)
    for u in sorted(out_cps):
        out_cps.pop(u).wait()

    n_tail = jnp.where(wid < _TAIL_GROUPS - 3 * _NW, 4, 3)

    def tail_body(k_loop, carry):
        t_base = 16 * (_MAIN_G + wid + _NW * k_loop)
        pltpu.sync_copy(tri_hbm.at[:, pl.ds(t_base, 16)],
                        idx_v[0].at[:, pl.ds(0, 16)])
        for k in range(3):
            pltpu.async_copy(
                table_hbm.at[idx_v[0].at[k, pl.ds(0, 16)]],
                verts_v[0][k].at[pl.ds(0, 16)], sem).wait()
        pltpu.sync_copy(bary_hbm.at[:, :, pl.ds(t_base, 16)],
                        bary_v[0].at[:, :, pl.ds(0, 16)])
        _compute_group(verts_v[0], bary_v[0], out_v[0], 0)
        pltpu.sync_copy(out_v[0].at[:, pl.ds(0, 96)],
                        out_hbm.at[:, pl.ds(6 * t_base, 96)])
        return carry

    lax.fori_loop(0, n_tail, tail_body, 0)


_mesh_kernel = pl.kernel(
    _mesh_body,
    out_type=jax.ShapeDtypeStruct((3, _T * 6), jnp.float32),
    mesh=plsc.VectorSubcoreMesh(core_axis_name="c", subcore_axis_name="s"),
    scratch_types=(
        [pltpu.VMEM((3, _C), jnp.int32)] * 2
        + [pltpu.VMEM((_C, 8), jnp.float32)] * 6
        + [pltpu.VMEM((6, 3, _C), jnp.float32)] * 2
        + [pltpu.VMEM((3, 6 * _C), jnp.float32)] * 2
        + [pltpu.SemaphoreType.DMA] * 3
    ),
    compiler_params=pltpu.CompilerParams(
        needs_layout_passes=False, use_tc_tiling_on_sc=False),
)


def kernel(vertex_coords, bary_coords, triangles):
    table8 = jnp.pad(vertex_coords, ((0, 0), (0, 5)))
    tri_t = jnp.transpose(triangles, (1, 0))
    bary_t = jnp.transpose(bary_coords, (1, 2, 0))
    out_t = _mesh_kernel(tri_t, bary_t, table8)
    return jnp.transpose(out_t, (1, 0))
